# Initial kernel scaffold; baseline (speedup 1.0000x reference)
import jax
import jax.numpy as jnp
from jax.experimental import pallas as pl


def kernel(x, edge_index, edge_label_index, W_conv, b_conv, W_lin, b_lin):
    raise NotImplementedError("write your pallas kernel here")


# speedup vs baseline: 13.7950x; 13.7950x over previous
import functools

import jax
import jax.numpy as jnp
from jax import lax
from jax.experimental import pallas as pl
from jax.experimental.pallas import tpu as pltpu
from jax.experimental.pallas import tpu_sc as plsc

N = 10000
E = 320000
L = 100000
D = 128
OUT = 128

NC = 2
NS = 16
NW = NC * NS
C = 128
EPW = E // NW
E_FULL = EPW // C
E_TAIL = EPW - E_FULL * C
RPS = 624
RTAIL = N - NS * RPS
LCHUNKS = (L + C - 1) // C
LPW = (LCHUNKS + NW - 1) // NW


def _mesh():
    return plsc.VectorSubcoreMesh(core_axis_name="c", subcore_axis_name="s")


def _sc_degree(dst, ones_rows, zerosD):
    @functools.partial(
        pl.kernel,
        mesh=_mesh(),
        out_type=jax.ShapeDtypeStruct((NC, N, D), jnp.float32),
        scratch_types=[
            pltpu.VMEM((C,), jnp.int32),
            pltpu.VMEM((E_TAIL,), jnp.int32),
            pltpu.VMEM((C, D), jnp.float32),
            pltpu.VMEM_SHARED((N, D), jnp.float32),
        ],
    )
    def k(dst_h, ones_h, zeros_h, out_h, idx_v, tidx_v, ones_v, deg_s):
        cid = lax.axis_index("c")
        sid = lax.axis_index("s")
        wid = sid * NC + cid
        pltpu.sync_copy(zeros_h.at[pl.ds(sid * RPS, RPS)],
                        deg_s.at[pl.ds(sid * RPS, RPS)])

        @pl.when(sid == 0)
        def _():
            pltpu.sync_copy(zeros_h.at[pl.ds(NS * RPS, RTAIL)],
                            deg_s.at[pl.ds(NS * RPS, RTAIL)])

        pltpu.sync_copy(ones_h, ones_v)
        plsc.subcore_barrier()
        base = wid * EPW

        def body(ci, carry):
            off = base + ci * C
            pltpu.sync_copy(dst_h.at[pl.ds(off, C)], idx_v)
            pltpu.sync_copy(ones_v, deg_s.at[idx_v], add=True)
            return carry

        lax.fori_loop(0, E_FULL, body, 0)
        toff = base + E_FULL * C
        pltpu.sync_copy(dst_h.at[pl.ds(toff, E_TAIL)], tidx_v)
        pltpu.sync_copy(ones_v.at[pl.ds(0, E_TAIL)], deg_s.at[tidx_v], add=True)
        plsc.subcore_barrier()
        pltpu.sync_copy(deg_s.at[pl.ds(sid * RPS, RPS)],
                        out_h.at[cid, pl.ds(sid * RPS, RPS)])

        @pl.when(sid == 0)
        def _():
            pltpu.sync_copy(deg_s.at[pl.ds(NS * RPS, RTAIL)],
                            out_h.at[cid, pl.ds(NS * RPS, RTAIL)])

    return k(dst, ones_rows, zerosD)


def _sc_messages(src, dst, y, zerosD):
    @functools.partial(
        pl.kernel,
        mesh=_mesh(),
        out_type=jax.ShapeDtypeStruct((NC, N, D), jnp.float32),
        scratch_types=[
            pltpu.VMEM((C,), jnp.int32),
            pltpu.VMEM((C,), jnp.int32),
            pltpu.VMEM((E_TAIL,), jnp.int32),
            pltpu.VMEM((E_TAIL,), jnp.int32),
            pltpu.VMEM((C, D), jnp.float32),
            pltpu.VMEM((E_TAIL, D), jnp.float32),
            pltpu.VMEM_SHARED((N, D), jnp.float32),
            pltpu.SemaphoreType.DMA,
        ],
    )
    def k(src_h, dst_h, y_h, zeros_h, out_h,
          sidx_v, didx_v, tsidx_v, tdidx_v, rows_v, trows_v, acc_s, sem):
        cid = lax.axis_index("c")
        sid = lax.axis_index("s")
        wid = sid * NC + cid
        pltpu.sync_copy(zeros_h.at[pl.ds(sid * RPS, RPS)],
                        acc_s.at[pl.ds(sid * RPS, RPS)])

        @pl.when(sid == 0)
        def _():
            pltpu.sync_copy(zeros_h.at[pl.ds(NS * RPS, RTAIL)],
                            acc_s.at[pl.ds(NS * RPS, RTAIL)])

        plsc.subcore_barrier()
        base = wid * EPW

        def body(ci, carry):
            off = base + ci * C
            pltpu.sync_copy(src_h.at[pl.ds(off, C)], sidx_v)
            pltpu.sync_copy(dst_h.at[pl.ds(off, C)], didx_v)
            pltpu.async_copy(y_h.at[sidx_v], rows_v, sem).wait()
            pltpu.sync_copy(rows_v, acc_s.at[didx_v], add=True)
            return carry

        lax.fori_loop(0, E_FULL, body, 0)
        toff = base + E_FULL * C
        pltpu.sync_copy(src_h.at[pl.ds(toff, E_TAIL)], tsidx_v)
        pltpu.sync_copy(dst_h.at[pl.ds(toff, E_TAIL)], tdidx_v)
        pltpu.async_copy(y_h.at[tsidx_v], trows_v, sem).wait()
        pltpu.sync_copy(trows_v, acc_s.at[tdidx_v], add=True)
        plsc.subcore_barrier()
        pltpu.sync_copy(acc_s.at[pl.ds(sid * RPS, RPS)],
                        out_h.at[cid, pl.ds(sid * RPS, RPS)])

        @pl.when(sid == 0)
        def _():
            pltpu.sync_copy(acc_s.at[pl.ds(NS * RPS, RTAIL)],
                            out_h.at[cid, pl.ds(NS * RPS, RTAIL)])

    return k(src, dst, y, zerosD)


def _sc_pairs(i0, i1, h1, h2):
    @functools.partial(
        pl.kernel,
        mesh=_mesh(),
        out_type=jax.ShapeDtypeStruct((L, OUT), jnp.float32),
        scratch_types=[
            pltpu.VMEM((C,), jnp.int32),
            pltpu.VMEM((C,), jnp.int32),
            pltpu.VMEM((C, OUT), jnp.float32),
            pltpu.VMEM((C, OUT), jnp.float32),
            pltpu.SemaphoreType.DMA,
            pltpu.SemaphoreType.DMA,
        ],
    )
    def k(i0_h, i1_h, h1_h, h2_h, out_h, i0_v, i1_v, ra_v, rb_v, sem_a, sem_b):
        cid = lax.axis_index("c")
        sid = lax.axis_index("s")
        wid = sid * NC + cid

        def body(kk, carry):
            c = wid + NW * kk

            @pl.when(c < LCHUNKS)
            def _():
                start = jnp.minimum(c * C, L - C)
                pltpu.sync_copy(i0_h.at[pl.ds(start, C)], i0_v)
                pltpu.sync_copy(i1_h.at[pl.ds(start, C)], i1_v)
                cp_a = pltpu.async_copy(h1_h.at[i0_v], ra_v, sem_a)
                cp_b = pltpu.async_copy(h2_h.at[i1_v], rb_v, sem_b)
                cp_a.wait()
                cp_b.wait()

                def add_row(r, cc):
                    for j in range(OUT // 16):
                        sl = pl.ds(j * 16, 16)
                        plsc.addupdate(ra_v.at[r, sl], rb_v[r, sl])
                    return cc

                lax.fori_loop(0, C, add_row, 0)
                pltpu.sync_copy(ra_v, out_h.at[pl.ds(start, C)])

            return carry

        lax.fori_loop(0, LPW, body, 0)

    return k(i0, i1, h1, h2)


_ROWS_BLK = 1000


def _tc_xw(x, w):
    def body(x_ref, w_ref, o_ref):
        o_ref[...] = jnp.dot(x_ref[...], w_ref[...],
                             preferred_element_type=jnp.float32)

    return pl.pallas_call(
        body,
        grid=(N // _ROWS_BLK,),
        in_specs=[pl.BlockSpec((_ROWS_BLK, D), lambda i: (i, 0)),
                  pl.BlockSpec((D, D), lambda i: (0, 0))],
        out_specs=pl.BlockSpec((_ROWS_BLK, D), lambda i: (i, 0)),
        out_shape=jax.ShapeDtypeStruct((N, D), jnp.float32),
    )(x, w)


def _tc_scale(degp, xw):
    def body(dp_ref, xw_ref, y_ref):
        dsum = dp_ref[0] + dp_ref[1]
        dis = lax.rsqrt(dsum[:, 0:1] + 1.0)
        y_ref[...] = dis * xw_ref[...]

    return pl.pallas_call(
        body,
        grid=(N // _ROWS_BLK,),
        in_specs=[pl.BlockSpec((NC, _ROWS_BLK, D), lambda i: (0, i, 0)),
                  pl.BlockSpec((_ROWS_BLK, D), lambda i: (i, 0))],
        out_specs=pl.BlockSpec((_ROWS_BLK, D), lambda i: (i, 0)),
        out_shape=jax.ShapeDtypeStruct((N, D), jnp.float32),
    )(degp, xw)


def _tc_post(degp, accp, y, W_lin, b_conv2, b_lin2):
    def body(dp_ref, acc_ref, y_ref, wl_ref, bc_ref, bl_ref, h1_ref, h2_ref):
        dsum = dp_ref[0] + dp_ref[1]
        dis = lax.rsqrt(dsum[:, 0:1] + 1.0)
        h = dis * (acc_ref[0] + acc_ref[1] + y_ref[...]) + bc_ref[...]
        h1_ref[...] = jnp.dot(h, wl_ref[0:D, :],
                              preferred_element_type=jnp.float32) + bl_ref[...]
        h2_ref[...] = jnp.dot(h, wl_ref[D:2 * D, :],
                              preferred_element_type=jnp.float32)

    return pl.pallas_call(
        body,
        grid=(N // _ROWS_BLK,),
        in_specs=[pl.BlockSpec((NC, _ROWS_BLK, D), lambda i: (0, i, 0)),
                  pl.BlockSpec((NC, _ROWS_BLK, D), lambda i: (0, i, 0)),
                  pl.BlockSpec((_ROWS_BLK, D), lambda i: (i, 0)),
                  pl.BlockSpec((2 * D, OUT), lambda i: (0, 0)),
                  pl.BlockSpec((1, D), lambda i: (0, 0)),
                  pl.BlockSpec((1, OUT), lambda i: (0, 0))],
        out_specs=[pl.BlockSpec((_ROWS_BLK, OUT), lambda i: (i, 0)),
                   pl.BlockSpec((_ROWS_BLK, OUT), lambda i: (i, 0))],
        out_shape=[jax.ShapeDtypeStruct((N, OUT), jnp.float32),
                   jax.ShapeDtypeStruct((N, OUT), jnp.float32)],
    )(degp, accp, y, W_lin, b_conv2, b_lin2)


def kernel(x, edge_index, edge_label_index, W_conv, b_conv, W_lin, b_lin):
    src = edge_index[0]
    dst = edge_index[1]
    i0 = edge_label_index[0]
    i1 = edge_label_index[1]
    ones_rows = jnp.ones((C, D), jnp.float32)
    zerosD = jnp.zeros((N, D), jnp.float32)

    degp = _sc_degree(dst, ones_rows, zerosD)
    xw = _tc_xw(x, W_conv)
    y = _tc_scale(degp, xw)
    accp = _sc_messages(src, dst, y, zerosD)
    h1, h2 = _tc_post(degp, accp, y, W_lin,
                      b_conv.reshape(1, D), b_lin.reshape(1, OUT))
    return _sc_pairs(i0, i1, h1, h2)
